# baseline (speedup 1.0000x reference)
import dataclasses
import functools

import jax
import jax.numpy as jnp
from jax import lax
from jax.experimental import pallas as pl
from jax.experimental.pallas import tpu as pltpu
from jax.experimental.pallas import tpu_sc as plsc

_MU = 3.5
_L = 16
_TW = 128
_SW = 256
_TR = 8
_HMAX = 1008
_RING = 16


def _iota():
    return lax.iota(jnp.int32, _L)


def _popcnt(m):
    return plsc.all_reduce_population_count(m)[0]


def _splat(h):
    return jnp.full((_L,), h, jnp.int32)


def _lfm_stage(B, K, V, NC, NS):
    NW = NC * NS
    nstr = V // _SW
    tailv = nstr * _SW
    spw = (nstr + NW - 1) // NW
    mesh = plsc.VectorSubcoreMesh(core_axis_name="c", subcore_axis_name="s")
    cp = pltpu.CompilerParams()
    if "needs_layout_passes" in pltpu.CompilerParams.__dataclass_fields__:
        cp = dataclasses.replace(cp, needs_layout_passes=False)

    @functools.partial(
        pl.kernel,
        out_type=(jax.ShapeDtypeStruct((B * K,), jnp.float32),
                  jax.ShapeDtypeStruct((B * K,), jnp.float32)),
        mesh=mesh,
        compiler_params=cp,
        scratch_types=[
            pltpu.VMEM((B,), jnp.int32),
            pltpu.VMEM((B,), jnp.int32),
            pltpu.VMEM((_HMAX + _L,), jnp.int32),
            pltpu.VMEM((_HMAX + _L,), jnp.int32),
            pltpu.VMEM((_HMAX + _L,), jnp.int32),
            pltpu.VMEM((_HMAX + _L,), jnp.int32),
            pltpu.VMEM((_HMAX + _L,), jnp.int32),
            pltpu.VMEM((_HMAX + _L,), jnp.int32),
            pltpu.VMEM((_HMAX + _L,), jnp.int32),
            pltpu.VMEM((_HMAX + _L,), jnp.int32),
            pltpu.VMEM((_HMAX + _L,), jnp.int32),
            pltpu.VMEM((_HMAX + _L,), jnp.int32),
            pltpu.VMEM((_HMAX + _L,), jnp.int32),
            pltpu.VMEM((_HMAX + _L,), jnp.int32),
            pltpu.VMEM((K, _SW), jnp.float32),
            pltpu.VMEM((K, _SW), jnp.float32),
            pltpu.VMEM((K, _SW), jnp.float32),
            pltpu.VMEM((K, _SW), jnp.float32),
            pltpu.VMEM((K, V - tailv), jnp.float32),
            pltpu.VMEM((K, V - tailv), jnp.float32),
            pltpu.VMEM((_RING * K,), jnp.float32),
            pltpu.VMEM((_RING * K,), jnp.float32),
            pltpu.SemaphoreType.DMA,
            pltpu.SemaphoreType.DMA,
            pltpu.SemaphoreType.DMA,
            pltpu.SemaphoreType.DMA,
            pltpu.SemaphoreType.DMA,
            pltpu.SemaphoreType.DMA,
            pltpu.SemaphoreType.DMA,
        ],
    )
    def ka(uidx_hbm, iidx_hbm, pt_hbm, qt_hbm, ps_hbm, qs_hbm,
           uall_v, iall_v, hu_v, hup_v, hi_v, hip_v,
           su_v, sup_v, si_v, sip_v,
           mu_v, mup_v, mi_v, mip_v,
           pb0, pb1, qb0, qb1, ptail, qtail, prb, qrb,
           sem_p0, sem_p1, sem_q0, sem_q1, sem_t, sem_wp, sem_wq):
        wid = lax.axis_index("s") * NC + lax.axis_index("c")
        s_lo = wid * spw
        s_hi = jnp.minimum(s_lo + spw, nstr)
        lo = s_lo * _SW
        hi = jnp.where(wid == NW - 1, V, s_hi * _SW)

        pltpu.sync_copy(uidx_hbm, uall_v)
        pltpu.sync_copy(iidx_hbm, iall_v)

        for a in range(K // _TR):
            pltpu.async_copy(
                pt_hbm.at[pl.ds(a * _TR, _TR), pl.ds(tailv, V - tailv)],
                ptail.at[pl.ds(a * _TR, _TR)], sem_t)
            pltpu.async_copy(
                qt_hbm.at[pl.ds(a * _TR, _TR), pl.ds(tailv, V - tailv)],
                qtail.at[pl.ds(a * _TR, _TR)], sem_t)

        def compact(src_v, dst_v, dstp_v, lo_, hi_):
            @pl.loop(0, B // _L, init_carry=jnp.int32(0))
            def off(g, off):
                x16 = src_v[pl.ds(g * _L, _L)]
                m = (x16 >= lo_) & (x16 < hi_) & (off <= _HMAX)
                plsc.store_compressed(dst_v.at[pl.ds(off, _L)], x16, mask=m)
                plsc.store_compressed(dstp_v.at[pl.ds(off, _L)],
                                      g * _L + _iota(), mask=m)
                return off + _popcnt(m)
            return off

        nu = compact(uall_v, hu_v, hup_v, lo, hi)
        ni = compact(iall_v, hi_v, hip_v, lo, hi)

        def strip_filter(n_hits, hv, hpv, dv, dpv, slo_, shi_):
            ng = (n_hits + (_L - 1)) // _L

            @pl.loop(0, ng, init_carry=jnp.int32(0))
            def cnt(g, off):
                x16 = hv[pl.ds(g * _L, _L)]
                m = ((x16 >= slo_) & (x16 < shi_)
                     & ((g * _L + _iota()) < n_hits))
                plsc.store_compressed(dv.at[pl.ds(off, _L)], x16, mask=m)
                p16 = hpv[pl.ds(g * _L, _L)]
                plsc.store_compressed(dpv.at[pl.ds(off, _L)], p16, mask=m)
                return off + _popcnt(m)
            return cnt

        def extract(nhit, dv, dpv, buf, cbase, rb, stage_hbm, sem_w, h0):
            @pl.loop(0, nhit, init_carry=h0)
            def hout(h, hc):
                u16 = plsc.load_gather(dv, [_splat(h)])
                p16 = plsc.load_gather(dpv, [_splat(h)])
                c = u16[0] - cbase
                pos = p16[0]
                slot = (hc & (_RING - 1)) * K
                for g4 in range(K // _L):
                    row16 = plsc.load_gather(
                        buf, [g4 * _L + _iota(), _splat(c)])
                    rb[pl.ds(slot + g4 * _L, _L)] = row16
                pltpu.async_copy(rb.at[pl.ds(slot, K)],
                                 stage_hbm.at[pl.ds(pos * K, K)], sem_w)

                @pl.when((hc & (_RING - 1)) == _RING - 1)
                def _():
                    @pl.loop(0, _RING)
                    def _(_k):
                        pltpu.make_async_copy(
                            stage_hbm.at[pl.ds(0, K)],
                            rb.at[pl.ds(0, K)], sem_w).wait()
                return hc + 1
            return hout

        def fire(s, pb, qb, sem_p, sem_q):
            pltpu.async_copy(pt_hbm.at[:, pl.ds(s * _SW, _SW)], pb, sem_p)
            pltpu.async_copy(qt_hbm.at[:, pl.ds(s * _SW, _SW)], qb, sem_q)

        def drain(pb, qb, sem_p, sem_q):
            pltpu.make_async_copy(
                pt_hbm.at[:, pl.ds(0, _SW)], pb, sem_p).wait()
            pltpu.make_async_copy(
                qt_hbm.at[:, pl.ds(0, _SW)], qb, sem_q).wait()

        def do_strip(s, pb, qb, hp, hq, nmu, nmi, drain_fn):
            slo_ = s * _SW
            shi_ = slo_ + _SW
            ns_u = strip_filter(nmu, mu_v, mup_v, su_v, sup_v, slo_, shi_)
            ns_i = strip_filter(nmi, mi_v, mip_v, si_v, sip_v, slo_, shi_)
            drain_fn()
            hp = extract(ns_u, su_v, sup_v, pb, slo_, prb, ps_hbm,
                         sem_wp, hp)
            hq = extract(ns_i, si_v, sip_v, qb, slo_, qrb, qs_hbm,
                         sem_wq, hq)
            return hp, hq

        fire(s_lo, pb0, qb0, sem_p0, sem_q0)

        @pl.loop(s_lo, s_hi, step=2,
                 init_carry=(jnp.int32(0), jnp.int32(0),
                             jnp.int32(0), jnp.int32(0)))
        def carry(s, hs):
            hp, hq, nmu, nmi = hs

            def refresh(nmu, nmi):
                mlo = s * _SW
                mhi = (s + 8) * _SW
                nmu = strip_filter(nu, hu_v, hup_v, mu_v, mup_v, mlo, mhi)
                nmi = strip_filter(ni, hi_v, hip_v, mi_v, mip_v, mlo, mhi)
                return nmu, nmi

            nmu, nmi = lax.cond(((s - s_lo) & 7) == 0, refresh,
                                lambda a, b: (a, b), nmu, nmi)

            @pl.when(s + 1 < s_hi)
            def _():
                fire(s + 1, pb1, qb1, sem_p1, sem_q1)

            hp, hq = do_strip(s, pb0, qb0, hp, hq, nmu, nmi,
                              lambda: drain(pb0, qb0, sem_p0, sem_q0))

            @pl.when(s + 2 < s_hi)
            def _():
                fire(s + 2, pb0, qb0, sem_p0, sem_q0)

            def odd(hp, hq):
                return do_strip(s + 1, pb1, qb1, hp, hq, nmu, nmi,
                                lambda: drain(pb1, qb1, sem_p1, sem_q1))

            hp, hq = lax.cond(s + 1 < s_hi, odd,
                              lambda hp, hq: (hp, hq), hp, hq)
            return hp, hq, nmu, nmi

        hp, hq, _nmu, _nmi = carry

        for a in range(K // _TR):
            pltpu.make_async_copy(
                pt_hbm.at[pl.ds(0, _TR), pl.ds(tailv, V - tailv)],
                ptail.at[pl.ds(a * _TR, _TR)], sem_t).wait()
            pltpu.make_async_copy(
                qt_hbm.at[pl.ds(0, _TR), pl.ds(tailv, V - tailv)],
                qtail.at[pl.ds(a * _TR, _TR)], sem_t).wait()
        ns_u = strip_filter(nu, hu_v, hup_v, su_v, sup_v, tailv, V)
        hp = extract(ns_u, su_v, sup_v, ptail, tailv, prb, ps_hbm,
                     sem_wp, hp)
        ns_i = strip_filter(ni, hi_v, hip_v, si_v, sip_v, tailv, V)
        hq = extract(ns_i, si_v, sip_v, qtail, tailv, qrb, qs_hbm,
                     sem_wq, hq)

        @pl.loop(0, hp & (_RING - 1))
        def _(_k):
            pltpu.make_async_copy(ps_hbm.at[pl.ds(0, K)],
                                  prb.at[pl.ds(0, K)], sem_wp).wait()

        @pl.loop(0, hq & (_RING - 1))
        def _(_k):
            pltpu.make_async_copy(qs_hbm.at[pl.ds(0, K)],
                                  qrb.at[pl.ds(0, K)], sem_wq).wait()

    return ka


def _lfm_score(B, K, NC, NS):
    NW = NC * NS
    bpw = B // NW
    mesh = plsc.VectorSubcoreMesh(core_axis_name="c", subcore_axis_name="s")
    cp = pltpu.CompilerParams()
    if "needs_layout_passes" in pltpu.CompilerParams.__dataclass_fields__:
        cp = dataclasses.replace(cp, needs_layout_passes=False)

    @functools.partial(
        pl.kernel,
        out_type=jax.ShapeDtypeStruct((B,), jnp.float32),
        mesh=mesh,
        compiler_params=cp,
        scratch_types=[
            pltpu.VMEM((bpw,), jnp.int32),
            pltpu.VMEM((bpw,), jnp.int32),
            pltpu.VMEM((bpw * K,), jnp.float32),
            pltpu.VMEM((bpw * K,), jnp.float32),
            pltpu.VMEM((bpw,), jnp.float32),
            pltpu.VMEM((bpw,), jnp.float32),
            pltpu.VMEM((bpw,), jnp.float32),
            pltpu.SemaphoreType.DMA,
            pltpu.SemaphoreType.DMA,
        ],
    )
    def kb(uidx_hbm, iidx_hbm, ps_hbm, qs_hbm, bu_hbm, bi_hbm, out_hbm,
           uidx_v, iidx_v, p_v, q_v, bu_v, bi_v, out_v, sem_bu, sem_bi):
        wid = lax.axis_index("s") * NC + lax.axis_index("c")
        base = wid * bpw

        pltpu.sync_copy(uidx_hbm.at[pl.ds(base, bpw)], uidx_v)
        pltpu.sync_copy(iidx_hbm.at[pl.ds(base, bpw)], iidx_v)
        cp_bu = pltpu.async_copy(bu_hbm.at[uidx_v], bu_v, sem_bu)
        cp_bi = pltpu.async_copy(bi_hbm.at[iidx_v], bi_v, sem_bi)
        pltpu.sync_copy(ps_hbm.at[pl.ds(base * K, bpw * K)], p_v)
        pltpu.sync_copy(qs_hbm.at[pl.ds(base * K, bpw * K)], q_v)

        @pl.loop(0, bpw, step=_L)
        def _(s0):
            rowbase = (s0 + _iota()) * K

            def body(j, acc):
                fidx = rowbase + j
                return acc + (plsc.load_gather(p_v, [fidx])
                              * plsc.load_gather(q_v, [fidx]))

            out_v[pl.ds(s0, _L)] = lax.fori_loop(
                0, K, body, jnp.zeros((_L,), jnp.float32))

        cp_bu.wait()
        cp_bi.wait()

        @pl.loop(0, bpw, step=_L)
        def _(s0):
            out_v[pl.ds(s0, _L)] = (
                out_v[pl.ds(s0, _L)] + bu_v[pl.ds(s0, _L)]
                + bi_v[pl.ds(s0, _L)] + _MU
            )

        pltpu.sync_copy(out_v, out_hbm.at[pl.ds(base, bpw)])

    return kb


def kernel(user_idx, item_idx, P, Q, b_u, b_i):
    B = user_idx.shape[0]
    V, K = P.shape
    info = plsc.get_sparse_core_info()
    ka = _lfm_stage(B, K, V, info.num_cores, info.num_subcores)
    kb = _lfm_score(B, K, info.num_cores, info.num_subcores)
    u32 = user_idx.astype(jnp.int32)
    i32 = item_idx.astype(jnp.int32)
    PT = jnp.swapaxes(P, 0, 1)
    QT = jnp.swapaxes(Q, 0, 1)
    p_stage, q_stage = ka(u32, i32, PT, QT)
    return kb(u32, i32, p_stage, q_stage, b_u, b_i)


# speedup vs baseline: 1.0303x; 1.0303x over previous
import dataclasses
import functools

import jax
import jax.numpy as jnp
from jax import lax
from jax.experimental import pallas as pl
from jax.experimental.pallas import tpu as pltpu
from jax.experimental.pallas import tpu_sc as plsc

_MU = 3.5
_L = 16
_TW = 128
_SW = 256
_TR = 8
_HMAX = 1008
_RING = 16


def _iota():
    return lax.iota(jnp.int32, _L)


def _popcnt(m):
    return plsc.all_reduce_population_count(m)[0]


def _splat(h):
    return jnp.full((_L,), h, jnp.int32)


def _lfm_stage(B, K, V, NC, NS):
    NW = NC * NS
    nstr = V // _SW
    tailv = nstr * _SW
    spw = (nstr + NW - 1) // NW
    mesh = plsc.VectorSubcoreMesh(core_axis_name="c", subcore_axis_name="s")
    cp = pltpu.CompilerParams()
    if "needs_layout_passes" in pltpu.CompilerParams.__dataclass_fields__:
        cp = dataclasses.replace(cp, needs_layout_passes=False)

    @functools.partial(
        pl.kernel,
        out_type=(jax.ShapeDtypeStruct((B * K,), jnp.float32),
                  jax.ShapeDtypeStruct((B * K,), jnp.float32)),
        mesh=mesh,
        compiler_params=cp,
        scratch_types=[
            pltpu.VMEM((B,), jnp.int32),
            pltpu.VMEM((B,), jnp.int32),
            pltpu.VMEM((_HMAX + _L,), jnp.int32),
            pltpu.VMEM((_HMAX + _L,), jnp.int32),
            pltpu.VMEM((_HMAX + _L,), jnp.int32),
            pltpu.VMEM((_HMAX + _L,), jnp.int32),
            pltpu.VMEM((_HMAX + _L,), jnp.int32),
            pltpu.VMEM((_HMAX + _L,), jnp.int32),
            pltpu.VMEM((_HMAX + _L,), jnp.int32),
            pltpu.VMEM((_HMAX + _L,), jnp.int32),
            pltpu.VMEM((K, _SW), jnp.float32),
            pltpu.VMEM((K, _SW), jnp.float32),
            pltpu.VMEM((K, _SW), jnp.float32),
            pltpu.VMEM((K, _SW), jnp.float32),
            pltpu.VMEM((K, V - tailv), jnp.float32),
            pltpu.VMEM((K, V - tailv), jnp.float32),
            pltpu.VMEM((_RING * K,), jnp.float32),
            pltpu.VMEM((_RING * K,), jnp.float32),
            pltpu.SemaphoreType.DMA,
            pltpu.SemaphoreType.DMA,
            pltpu.SemaphoreType.DMA,
            pltpu.SemaphoreType.DMA,
            pltpu.SemaphoreType.DMA,
            pltpu.SemaphoreType.DMA,
            pltpu.SemaphoreType.DMA,
        ],
    )
    def ka(uidx_hbm, iidx_hbm, pt_hbm, qt_hbm, ps_hbm, qs_hbm,
           uall_v, iall_v, hu_v, hup_v, hi_v, hip_v,
           su_v, sup_v, si_v, sip_v,
           pb0, pb1, qb0, qb1, ptail, qtail, prb, qrb,
           sem_p0, sem_p1, sem_q0, sem_q1, sem_t, sem_wp, sem_wq):
        wid = lax.axis_index("s") * NC + lax.axis_index("c")
        s_lo = wid * spw
        s_hi = jnp.minimum(s_lo + spw, nstr)
        lo = s_lo * _SW
        hi = jnp.where(wid == NW - 1, V, s_hi * _SW)

        pltpu.sync_copy(uidx_hbm, uall_v)
        pltpu.sync_copy(iidx_hbm, iall_v)

        for a in range(K // _TR):
            pltpu.async_copy(
                pt_hbm.at[pl.ds(a * _TR, _TR), pl.ds(tailv, V - tailv)],
                ptail.at[pl.ds(a * _TR, _TR)], sem_t)
            pltpu.async_copy(
                qt_hbm.at[pl.ds(a * _TR, _TR), pl.ds(tailv, V - tailv)],
                qtail.at[pl.ds(a * _TR, _TR)], sem_t)

        def compact(src_v, dst_v, dstp_v, lo_, hi_):
            @pl.loop(0, B // _L, init_carry=jnp.int32(0))
            def off(g, off):
                x16 = src_v[pl.ds(g * _L, _L)]
                m = (x16 >= lo_) & (x16 < hi_) & (off <= _HMAX)
                plsc.store_compressed(dst_v.at[pl.ds(off, _L)], x16, mask=m)
                plsc.store_compressed(dstp_v.at[pl.ds(off, _L)],
                                      g * _L + _iota(), mask=m)
                return off + _popcnt(m)
            return off

        nu = compact(uall_v, hu_v, hup_v, lo, hi)
        ni = compact(iall_v, hi_v, hip_v, lo, hi)

        def strip_filter(n_hits, hv, hpv, dv, dpv, slo_, shi_):
            ng = (n_hits + (_L - 1)) // _L

            @pl.loop(0, ng, init_carry=jnp.int32(0))
            def cnt(g, off):
                x16 = hv[pl.ds(g * _L, _L)]
                m = ((x16 >= slo_) & (x16 < shi_)
                     & ((g * _L + _iota()) < n_hits))
                plsc.store_compressed(dv.at[pl.ds(off, _L)], x16, mask=m)
                p16 = hpv[pl.ds(g * _L, _L)]
                plsc.store_compressed(dpv.at[pl.ds(off, _L)], p16, mask=m)
                return off + _popcnt(m)
            return cnt

        def extract(nhit, dv, dpv, buf, cbase, rb, stage_hbm, sem_w, h0):
            @pl.loop(0, nhit, init_carry=h0)
            def hout(h, hc):
                u16 = plsc.load_gather(dv, [_splat(h)])
                p16 = plsc.load_gather(dpv, [_splat(h)])
                c = u16[0] - cbase
                pos = p16[0]
                slot = (hc & (_RING - 1)) * K
                for g4 in range(K // _L):
                    row16 = plsc.load_gather(
                        buf, [g4 * _L + _iota(), _splat(c)])
                    rb[pl.ds(slot + g4 * _L, _L)] = row16
                pltpu.async_copy(rb.at[pl.ds(slot, K)],
                                 stage_hbm.at[pl.ds(pos * K, K)], sem_w)

                @pl.when((hc & (_RING - 1)) == _RING - 1)
                def _():
                    @pl.loop(0, _RING)
                    def _(_k):
                        pltpu.make_async_copy(
                            stage_hbm.at[pl.ds(0, K)],
                            rb.at[pl.ds(0, K)], sem_w).wait()
                return hc + 1
            return hout

        def fire(s, pb, qb, sem_p, sem_q):
            pltpu.async_copy(pt_hbm.at[:, pl.ds(s * _SW, _SW)], pb, sem_p)
            pltpu.async_copy(qt_hbm.at[:, pl.ds(s * _SW, _SW)], qb, sem_q)

        def drain(pb, qb, sem_p, sem_q):
            pltpu.make_async_copy(
                pt_hbm.at[:, pl.ds(0, _SW)], pb, sem_p).wait()
            pltpu.make_async_copy(
                qt_hbm.at[:, pl.ds(0, _SW)], qb, sem_q).wait()

        def do_strip(s, pb, qb, hp, hq, drain_fn):
            slo_ = s * _SW
            shi_ = slo_ + _SW
            ns_u = strip_filter(nu, hu_v, hup_v, su_v, sup_v, slo_, shi_)
            ns_i = strip_filter(ni, hi_v, hip_v, si_v, sip_v, slo_, shi_)
            drain_fn()
            hp = extract(ns_u, su_v, sup_v, pb, slo_, prb, ps_hbm,
                         sem_wp, hp)
            hq = extract(ns_i, si_v, sip_v, qb, slo_, qrb, qs_hbm,
                         sem_wq, hq)
            return hp, hq

        fire(s_lo, pb0, qb0, sem_p0, sem_q0)

        @pl.loop(s_lo, s_hi, step=2,
                 init_carry=(jnp.int32(0), jnp.int32(0)))
        def carry(s, hs):
            hp, hq = hs

            @pl.when(s + 1 < s_hi)
            def _():
                fire(s + 1, pb1, qb1, sem_p1, sem_q1)

            hp, hq = do_strip(s, pb0, qb0, hp, hq,
                              lambda: drain(pb0, qb0, sem_p0, sem_q0))

            @pl.when(s + 2 < s_hi)
            def _():
                fire(s + 2, pb0, qb0, sem_p0, sem_q0)

            def odd(hp, hq):
                return do_strip(s + 1, pb1, qb1, hp, hq,
                                lambda: drain(pb1, qb1, sem_p1, sem_q1))

            hp, hq = lax.cond(s + 1 < s_hi, odd,
                              lambda hp, hq: (hp, hq), hp, hq)
            return hp, hq

        hp, hq = carry

        for a in range(K // _TR):
            pltpu.make_async_copy(
                pt_hbm.at[pl.ds(0, _TR), pl.ds(tailv, V - tailv)],
                ptail.at[pl.ds(a * _TR, _TR)], sem_t).wait()
            pltpu.make_async_copy(
                qt_hbm.at[pl.ds(0, _TR), pl.ds(tailv, V - tailv)],
                qtail.at[pl.ds(a * _TR, _TR)], sem_t).wait()
        ns_u = strip_filter(nu, hu_v, hup_v, su_v, sup_v, tailv, V)
        hp = extract(ns_u, su_v, sup_v, ptail, tailv, prb, ps_hbm,
                     sem_wp, hp)
        ns_i = strip_filter(ni, hi_v, hip_v, si_v, sip_v, tailv, V)
        hq = extract(ns_i, si_v, sip_v, qtail, tailv, qrb, qs_hbm,
                     sem_wq, hq)

        @pl.loop(0, hp & (_RING - 1))
        def _(_k):
            pltpu.make_async_copy(ps_hbm.at[pl.ds(0, K)],
                                  prb.at[pl.ds(0, K)], sem_wp).wait()

        @pl.loop(0, hq & (_RING - 1))
        def _(_k):
            pltpu.make_async_copy(qs_hbm.at[pl.ds(0, K)],
                                  qrb.at[pl.ds(0, K)], sem_wq).wait()

    return ka


def _lfm_score(B, K, NC, NS):
    NW = NC * NS
    bpw = B // NW
    mesh = plsc.VectorSubcoreMesh(core_axis_name="c", subcore_axis_name="s")
    cp = pltpu.CompilerParams()
    if "needs_layout_passes" in pltpu.CompilerParams.__dataclass_fields__:
        cp = dataclasses.replace(cp, needs_layout_passes=False)

    @functools.partial(
        pl.kernel,
        out_type=jax.ShapeDtypeStruct((B,), jnp.float32),
        mesh=mesh,
        compiler_params=cp,
        scratch_types=[
            pltpu.VMEM((bpw,), jnp.int32),
            pltpu.VMEM((bpw,), jnp.int32),
            pltpu.VMEM((bpw * K,), jnp.float32),
            pltpu.VMEM((bpw * K,), jnp.float32),
            pltpu.VMEM((bpw,), jnp.float32),
            pltpu.VMEM((bpw,), jnp.float32),
            pltpu.VMEM((bpw,), jnp.float32),
            pltpu.SemaphoreType.DMA,
            pltpu.SemaphoreType.DMA,
        ],
    )
    def kb(uidx_hbm, iidx_hbm, ps_hbm, qs_hbm, bu_hbm, bi_hbm, out_hbm,
           uidx_v, iidx_v, p_v, q_v, bu_v, bi_v, out_v, sem_bu, sem_bi):
        wid = lax.axis_index("s") * NC + lax.axis_index("c")
        base = wid * bpw

        pltpu.sync_copy(uidx_hbm.at[pl.ds(base, bpw)], uidx_v)
        pltpu.sync_copy(iidx_hbm.at[pl.ds(base, bpw)], iidx_v)
        cp_bu = pltpu.async_copy(bu_hbm.at[uidx_v], bu_v, sem_bu)
        cp_bi = pltpu.async_copy(bi_hbm.at[iidx_v], bi_v, sem_bi)
        pltpu.sync_copy(ps_hbm.at[pl.ds(base * K, bpw * K)], p_v)
        pltpu.sync_copy(qs_hbm.at[pl.ds(base * K, bpw * K)], q_v)

        @pl.loop(0, bpw, step=_L)
        def _(s0):
            rowbase = (s0 + _iota()) * K

            def body(j, acc):
                fidx = rowbase + j
                return acc + (plsc.load_gather(p_v, [fidx])
                              * plsc.load_gather(q_v, [fidx]))

            out_v[pl.ds(s0, _L)] = lax.fori_loop(
                0, K, body, jnp.zeros((_L,), jnp.float32))

        cp_bu.wait()
        cp_bi.wait()

        @pl.loop(0, bpw, step=_L)
        def _(s0):
            out_v[pl.ds(s0, _L)] = (
                out_v[pl.ds(s0, _L)] + bu_v[pl.ds(s0, _L)]
                + bi_v[pl.ds(s0, _L)] + _MU
            )

        pltpu.sync_copy(out_v, out_hbm.at[pl.ds(base, bpw)])

    return kb


def kernel(user_idx, item_idx, P, Q, b_u, b_i):
    B = user_idx.shape[0]
    V, K = P.shape
    info = plsc.get_sparse_core_info()
    ka = _lfm_stage(B, K, V, info.num_cores, info.num_subcores)
    kb = _lfm_score(B, K, info.num_cores, info.num_subcores)
    u32 = user_idx.astype(jnp.int32)
    i32 = item_idx.astype(jnp.int32)
    PT = jnp.swapaxes(P, 0, 1)
    QT = jnp.swapaxes(Q, 0, 1)
    p_stage, q_stage = ka(u32, i32, PT, QT)
    return kb(u32, i32, p_stage, q_stage, b_u, b_i)
